# baseline (speedup 1.0000x reference)
import functools

import jax
import jax.numpy as jnp
from jax import lax
from jax.experimental import pallas as pl
from jax.experimental.pallas import tpu as pltpu
from jax.experimental.pallas import tpu_sc as plsc

N = 10000
NP = 10240
E = 320000
D = 128
G = 64
NC = 2
NS = 16
NW = NC * NS
EPT = E // NW
CH = 80
NCH = EPT // CH
RPT = NP // NS

HI = NP // D
EB = 2000
NEB = E // EB

BLK = 2000
NB = N // BLK


def _sc_agg_body(x_hbm, zip_hbm, out_sum,
                 acc, sd0, sd1, sd2, rows0, rows1, rows2,
                 sem_ga, sem_gb, sem_sa, sem_sb):
    cid = lax.axis_index("c")
    sid = lax.axis_index("s")
    w = cid * NS + sid

    z16 = jnp.zeros((16,), jnp.float32)

    @pl.loop(0, CH)
    def _(i):
        for j in range(D // 16):
            rows0[i, pl.ds(j * 16, 16)] = z16

    @pl.loop(0, RPT // CH)
    def _(k):
        z = pl.multiple_of(sid * RPT + k * CH, 8)
        pltpu.async_copy(rows0, acc.at[pl.ds(z, CH)], sem_sa).wait()

    plsc.subcore_barrier()

    g0 = w * NCH
    sd = (sd0, sd1, sd2)
    rows = (rows0, rows1, rows2)
    gsem = (sem_ga, sem_gb)
    ssem = (sem_sa, sem_sb)

    def load_idx(c, cur, par):
        pltpu.async_copy(zip_hbm.at[g0 + c], sd[cur], gsem[par]).wait()

    def fire_gather(c, cur, par):
        pltpu.async_copy(x_hbm.at[sd[cur].at[0]], rows[cur], gsem[par])

    def wait_gather(c, cur, par):
        pltpu.make_async_copy(x_hbm.at[sd[cur].at[0]], rows[cur],
                              gsem[par]).wait()

    def fire_scatter(c, cur, par):
        pltpu.async_copy(rows[cur], acc.at[sd[cur].at[1]], ssem[par],
                         add=True)

    def drain_scatter(c, cur, par):
        pltpu.make_async_copy(rows[cur], acc.at[sd[cur].at[1]],
                              ssem[par]).wait()

    load_idx(0, 0, 0)
    fire_gather(0, 0, 0)

    def phase(c, p, guard_i=None):
        cur = (1 + p) % 3
        cpar = (1 + p) % 2
        prev = p % 3
        ppar = p % 2
        if guard_i is None:
            drain_scatter(c - 3, cur, ppar)
        else:
            @pl.when(guard_i > 0)
            def _():
                drain_scatter(c - 3, cur, ppar)
        load_idx(c, cur, cpar)
        fire_gather(c, cur, cpar)
        wait_gather(c - 1, prev, ppar)
        fire_scatter(c - 1, prev, ppar)

    @pl.loop(0, (NCH - 5) // 6)
    def _(i):
        for p in range(6):
            phase(6 * i + 1 + p, p, i if p < 2 else None)

    for p in range(4):
        phase(121 + p, p)
    wait_gather(124, 1, 0)
    fire_scatter(124, 1, 0)
    drain_scatter(122, 2, 0)
    drain_scatter(123, 0, 1)
    drain_scatter(124, 1, 0)

    plsc.subcore_barrier()

    r0 = pl.multiple_of(sid * RPT, 8)
    pltpu.async_copy(acc.at[pl.ds(r0, RPT)],
                     out_sum.at[cid, pl.ds(r0, RPT)], sem_sa).wait()


_sc_agg = functools.partial(
    pl.kernel,
    out_type=jax.ShapeDtypeStruct((NC, NP, D), jnp.float32),
    mesh=plsc.VectorSubcoreMesh(core_axis_name="c", subcore_axis_name="s"),
    scratch_types=[
        pltpu.VMEM_SHARED((NP, D), jnp.float32),
        pltpu.VMEM((2, CH), jnp.int32),
        pltpu.VMEM((2, CH), jnp.int32),
        pltpu.VMEM((2, CH), jnp.int32),
        pltpu.VMEM((CH, D), jnp.float32),
        pltpu.VMEM((CH, D), jnp.float32),
        pltpu.VMEM((CH, D), jnp.float32),
    ] + [pltpu.SemaphoreType.DMA] * 4,
)(_sc_agg_body)


def _cnt_body(dst_ref, cnt_ref):
    i = pl.program_id(0)

    @pl.when(i == 0)
    def _():
        cnt_ref[...] = jnp.zeros_like(cnt_ref)

    d = dst_ref[...]
    hi = d // D
    lo = d - hi * D
    ahi = (hi == lax.broadcasted_iota(jnp.int32, (EB, HI), 1)
           ).astype(jnp.bfloat16)
    alo = (lo == lax.broadcasted_iota(jnp.int32, (EB, D), 1)
           ).astype(jnp.bfloat16)
    cnt_ref[...] += lax.dot_general(
        ahi, alo, (((0,), (0,)), ((), ())),
        preferred_element_type=jnp.float32)


def _tc_cnt(dst2):
    return pl.pallas_call(
        _cnt_body,
        grid=(NEB,),
        in_specs=[pl.BlockSpec((EB, 1), lambda i: (i, 0))],
        out_specs=pl.BlockSpec((HI, D), lambda i: (0, 0)),
        out_shape=jax.ShapeDtypeStruct((HI, D), jnp.float32),
    )(dst2)


def _tc_body(sum_ref, cnt_ref, x_ref, bvec_ref, bsmem_ref,
             wlT_ref, wrT_ref, bl_ref, wlinT_ref, blin_ref,
             out_ref, acc_ref):
    i = pl.program_id(0)

    @pl.when(i == 0)
    def _():
        acc_ref[...] = jnp.zeros_like(acc_ref)

    summed = sum_ref[0] + sum_ref[1]
    cnt = cnt_ref[...]
    mean = summed / jnp.maximum(cnt, 1.0)
    h = (jnp.dot(mean, wlT_ref[...], preferred_element_type=jnp.float32)
         + jnp.dot(x_ref[...], wrT_ref[...], preferred_element_type=jnp.float32)
         + bl_ref[...])
    h = jnp.maximum(h, 0.0)

    bvec = bvec_ref[...]
    g0 = bsmem_ref[i * BLK]
    g1 = bsmem_ref[i * BLK + BLK - 1]
    rows_iota = lax.broadcasted_iota(jnp.int32, (G, 1), 0)

    def pool_body(g, c):
        m = bvec == g
        contrib = jnp.max(jnp.where(m, h, 0.0), axis=0, keepdims=True)
        upd = jnp.where(rows_iota == g, contrib, 0.0)
        acc_ref[...] = jnp.maximum(acc_ref[...], upd)
        return c
    lax.fori_loop(g0, g1 + 1, pool_body, 0)

    @pl.when(i == NB - 1)
    def _():
        out_ref[...] = (
            jnp.dot(acc_ref[...], wlinT_ref[...],
                    preferred_element_type=jnp.float32)
            + blin_ref[...])


def _tc_finish(parts_sum, cnt, x, bvec, batch, wlT, wrT, bl2,
               wlinT, blin2):
    return pl.pallas_call(
        _tc_body,
        grid=(NB,),
        in_specs=[
            pl.BlockSpec((NC, BLK, D), lambda i: (0, i, 0)),
            pl.BlockSpec((BLK, 1), lambda i: (i, 0)),
            pl.BlockSpec((BLK, D), lambda i: (i, 0)),
            pl.BlockSpec((BLK, 1), lambda i: (i, 0)),
            pl.BlockSpec(memory_space=pltpu.SMEM),
            pl.BlockSpec((D, D), lambda i: (0, 0)),
            pl.BlockSpec((D, D), lambda i: (0, 0)),
            pl.BlockSpec((1, D), lambda i: (0, 0)),
            pl.BlockSpec((D, 2), lambda i: (0, 0)),
            pl.BlockSpec((1, 2), lambda i: (0, 0)),
        ],
        out_specs=pl.BlockSpec((G, 2), lambda i: (0, 0)),
        out_shape=jax.ShapeDtypeStruct((G, 2), jnp.float32),
        scratch_shapes=[pltpu.VMEM((G, D), jnp.float32)],
    )(parts_sum, cnt, x, bvec, batch, wlT, wrT, bl2, wlinT, blin2)


def kernel(x, edge_index, batch, Wl, bl, Wr, Wlin, blin):
    dst = edge_index[1]
    ezip = jnp.stack([edge_index[0].reshape(NW * NCH, CH),
                      dst.reshape(NW * NCH, CH)], axis=1)
    parts_sum = _sc_agg(x, ezip)
    cnt2d = _tc_cnt(dst.reshape(E, 1))
    cnt = cnt2d.reshape(NP)[:N].reshape(N, 1)
    return _tc_finish(parts_sum, cnt, x, batch.reshape(N, 1), batch,
                      Wl.T, Wr.T, bl.reshape(1, D), Wlin.T,
                      blin.reshape(1, 2))
